# baseline (speedup 1.0000x reference)
import jax
import jax.numpy as jnp
from jax.experimental import pallas as pl

HIDDEN = 2048


def kernel(image_embeddings, text_embeddings, vision_indices):
    img = image_embeddings.reshape(-1, HIDDEN)
    flat = text_embeddings.reshape(-1, HIDDEN)
    n = vision_indices.shape[0]
    img = img[:n]
    j = jnp.arange(n, dtype=jnp.int32)
    winners = jnp.full((flat.shape[0],), -1, jnp.int32).at[vision_indices].max(j)
    out = jnp.where((winners >= 0)[:, None], img[jnp.maximum(winners, 0)], flat)
    return out.reshape(text_embeddings.shape)


# speedup vs baseline: 2.3755x; 2.3755x over previous
import functools

import jax
import jax.numpy as jnp
from jax import lax
from jax.experimental import pallas as pl
from jax.experimental.pallas import tpu as pltpu
from jax.experimental.pallas import tpu_sc as plsc

HIDDEN = 2048
N_IDX = 4096
N_ROWS = 16384

_INFO = plsc.get_sparse_core_info()
_NC = _INFO.num_cores
_NS = _INFO.num_subcores
_NW = _NC * _NS
_JPW = N_IDX // _NW
_CH = 32
_NCH = _JPW // _CH
_LANES = 16


def _sc_body(img_hbm, idx_hbm, out_ref, idx_v, m_v, tgt_v, src_v, rows_v, sem):
    wid = lax.axis_index("s") * _NC + lax.axis_index("c")
    base = wid * _JPW

    pltpu.sync_copy(idx_hbm, idx_v)

    lane = lax.iota(jnp.int32, _LANES)

    @pl.loop(0, N_IDX // _LANES, unroll=2)
    def _(k):
        off = k * _LANES
        tv = idx_v[pl.ds(off, _LANES)]
        key = tv * _LANES + lane
        skey = plsc.sort_key_val(key, key)[0]
        sidx = lax.shift_right_logical(skey, 4)
        sj = off + lax.rem(skey, _LANES)
        c = sj - sidx * (N_IDX * 2)
        win = jnp.flip(plsc.cummax(jnp.flip(c))) + sidx * (N_IDX * 2)
        plsc.store_scatter(m_v, [sidx], win)

    for c in range(_NCH):
        for l in range(_CH // _LANES):
            off = base + c * _CH + l * _LANES
            tv = idx_v[pl.ds(off, _LANES)]
            sv = plsc.load_gather(m_v, [tv])
            tgt_v[c, pl.ds(l * _LANES, _LANES)] = tv
            src_v[c, pl.ds(l * _LANES, _LANES)] = sv

    for c in range(_NCH):
        pltpu.async_copy(img_hbm.at[src_v.at[c]], rows_v, sem).wait()
        pltpu.async_copy(rows_v, out_ref.at[tgt_v.at[c]], sem).wait()


_sc_scatter = functools.partial(
    pl.kernel,
    mesh=plsc.VectorSubcoreMesh(core_axis_name="c", subcore_axis_name="s"),
    compiler_params=pltpu.CompilerParams(needs_layout_passes=False),
    scratch_types=[
        pltpu.VMEM((N_IDX,), jnp.int32),
        pltpu.VMEM((N_ROWS,), jnp.int32),
        pltpu.VMEM((_NCH, _CH), jnp.int32),
        pltpu.VMEM((_NCH, _CH), jnp.int32),
        pltpu.VMEM((_CH, HIDDEN), jnp.float32),
        pltpu.SemaphoreType.DMA,
    ],
)(_sc_body)


def kernel(image_embeddings, text_embeddings, vision_indices):
    flat = text_embeddings.reshape(-1, HIDDEN)
    img = image_embeddings.reshape(-1, HIDDEN)[:N_IDX]
    img = img.astype(text_embeddings.dtype)
    out_ref = jax.new_ref(flat)
    _sc_scatter(img, vision_indices.astype(jnp.int32), out_ref)
    return out_ref[...].reshape(text_embeddings.shape)


# speedup vs baseline: 2.3876x; 1.0051x over previous
import functools

import jax
import jax.numpy as jnp
from jax import lax
from jax.experimental import pallas as pl
from jax.experimental.pallas import tpu as pltpu
from jax.experimental.pallas import tpu_sc as plsc

HIDDEN = 2048
N_IDX = 4096
N_ROWS = 16384

_INFO = plsc.get_sparse_core_info()
_NC = _INFO.num_cores
_NS = _INFO.num_subcores
_NW = _NC * _NS
_JPW = N_IDX // _NW
_CH = 16
_NCH = _JPW // _CH
_NBUF = 3
_LANES = 16


def _sc_body(img_hbm, idx_hbm, out_ref, idx_v, m_v, tgt_v, src_v,
             rows0, rows1, sem_g, sem_s):
    bufs = (rows0, rows1)
    wid = lax.axis_index("s") * _NC + lax.axis_index("c")
    base = wid * _JPW

    pltpu.sync_copy(idx_hbm, idx_v)

    lane = lax.iota(jnp.int32, _LANES)

    @pl.loop(0, N_IDX // _LANES, unroll=2)
    def _(k):
        off = k * _LANES
        tv = idx_v[pl.ds(off, _LANES)]
        key = tv * _LANES + lane
        skey = plsc.sort_key_val(key, key)[0]
        sidx = lax.shift_right_logical(skey, 4)
        sj = off + lax.rem(skey, _LANES)
        c = sj - sidx * (N_IDX * 2)
        win = jnp.flip(plsc.cummax(jnp.flip(c))) + sidx * (N_IDX * 2)
        plsc.store_scatter(m_v, [sidx], win)

    for c in range(_NCH):
        for l in range(_CH // _LANES):
            off = base + c * _CH + l * _LANES
            tv = idx_v[pl.ds(off, _LANES)]
            sv = plsc.load_gather(m_v, [tv])
            tgt_v[c, pl.ds(l * _LANES, _LANES)] = tv
            src_v[c, pl.ds(l * _LANES, _LANES)] = sv

    def gather(c):
        return pltpu.async_copy(
            img_hbm.at[src_v.at[c]], bufs[c % 2], sem_g)

    def scatter(c):
        return pltpu.async_copy(
            bufs[c % 2], out_ref.at[tgt_v.at[c]], sem_s)

    g = gather(0)
    for c in range(_NCH):
        g.wait()
        s = scatter(c)
        if c + 1 < _NCH:
            g = gather(c + 1)
        s.wait()


_sc_scatter = functools.partial(
    pl.kernel,
    mesh=plsc.VectorSubcoreMesh(core_axis_name="c", subcore_axis_name="s"),
    compiler_params=pltpu.CompilerParams(needs_layout_passes=False),
    scratch_types=[
        pltpu.VMEM((N_IDX,), jnp.int32),
        pltpu.VMEM((N_ROWS,), jnp.int32),
        pltpu.VMEM((_NCH, _CH), jnp.int32),
        pltpu.VMEM((_NCH, _CH), jnp.int32),
        pltpu.VMEM((_CH, HIDDEN), jnp.float32),
        pltpu.VMEM((_CH, HIDDEN), jnp.float32),
        pltpu.SemaphoreType.DMA,
        pltpu.SemaphoreType.DMA,
    ],
)(_sc_body)


def kernel(image_embeddings, text_embeddings, vision_indices):
    flat = text_embeddings.reshape(-1, HIDDEN)
    img = image_embeddings.reshape(-1, HIDDEN)[:N_IDX]
    img = img.astype(text_embeddings.dtype)
    out_ref = jax.new_ref(flat)
    _sc_scatter(img, vision_indices.astype(jnp.int32), out_ref)
    return out_ref[...].reshape(text_embeddings.shape)
